# baseline (speedup 1.0000x reference)
import functools

import jax
import jax.numpy as jnp
from jax import lax
from jax.experimental import pallas as pl
from jax.experimental.pallas import tpu as pltpu
from jax.experimental.pallas import tpu_sc as plsc

LR = 0.01
EPS = 1e-10
M = 1000000
D = 32
B = 16384

NS = 16
C = B // NS
SB = 128
NB = C // SB
L = 16
RPT = M // NS


def _rsqrt(s):
    s = jnp.maximum(s, 1e-37)
    i = lax.bitcast_convert_type(s, jnp.int32)
    y = lax.bitcast_convert_type(jnp.int32(0x5F3759DF) - (i >> 1),
                                 jnp.float32)
    for _ in range(3):
        y = y * (1.5 - 0.5 * s * y * y)
    return s, y


def _sc_body(idx_hbm, grad_hbm, emb_hbm, state_hbm, out_emb, out_state,
             slot_tab, gsum, cntt,
             idx_v, rep_v, gbuf, obuf, rowbuf, sem, csem):
    tid = lax.axis_index("s")
    base = tid * C
    cbase = tid * RPT

    zeros16 = jnp.zeros((L,), jnp.float32)
    ones16 = jnp.ones((L,), jnp.float32)
    iota16 = lax.iota(jnp.int32, L)

    copy_descs = [
        pltpu.async_copy(emb_hbm.at[pl.ds(cbase, RPT)],
                         out_emb.at[pl.ds(cbase, RPT)], csem),
        pltpu.async_copy(state_hbm.at[pl.ds(cbase, RPT)],
                         out_state.at[pl.ds(cbase, RPT)], csem),
    ]

    pltpu.sync_copy(idx_hbm.at[pl.ds(tid * NB, NB)], idx_v)
    for j in range(NB):
        def _jfill(k, carry, j=j):
            rep_v[j, pl.ds(k * L, L)] = iota16 + (base + j * SB + k * L)
            return carry
        lax.fori_loop(0, SB // L, _jfill, 0)
        pltpu.sync_copy(rep_v.at[j], slot_tab.at[idx_v.at[j]])

    def _zrow(r, carry):
        gbuf[r, pl.ds(0, L)] = zeros16
        gbuf[r, pl.ds(L, L)] = zeros16
        obuf[r] = zeros16
        return carry
    lax.fori_loop(0, SB, _zrow, 0)
    for j in range(NB):
        pltpu.sync_copy(gbuf, gsum.at[pl.ds(base + j * SB, SB)])
        pltpu.sync_copy(obuf, cntt.at[pl.ds(base + j * SB, SB)])

    plsc.subcore_barrier()

    descs = [pltpu.async_copy(slot_tab.at[idx_v.at[j]], rep_v.at[j], sem)
             for j in range(NB)]
    for d in descs:
        d.wait()

    def _orow(r, carry):
        obuf[r] = ones16
        return carry
    lax.fori_loop(0, SB, _orow, 0)

    for j in range(NB):
        pltpu.sync_copy(grad_hbm.at[pl.ds(base + j * SB, SB)], gbuf)
        pltpu.sync_copy(gbuf, gsum.at[rep_v.at[j]], add=True)
        pltpu.sync_copy(obuf, cntt.at[rep_v.at[j]], add=True)

    for d in copy_descs:
        d.wait()
    plsc.subcore_barrier()

    for j in range(NB):
        pltpu.sync_copy(gsum.at[rep_v.at[j]], gbuf)
        pltpu.sync_copy(cntt.at[rep_v.at[j]], obuf)
        pltpu.sync_copy(state_hbm.at[idx_v.at[j]], rowbuf)

        def _srow(r, carry):
            cnt = obuf[r]
            m0 = gbuf[r, pl.ds(0, L)] / cnt
            m1 = gbuf[r, pl.ds(L, L)] / cnt
            s0 = rowbuf[r, pl.ds(0, L)] + m0 * m0
            s1 = rowbuf[r, pl.ds(L, L)] + m1 * m1
            rowbuf[r, pl.ds(0, L)] = s0
            rowbuf[r, pl.ds(L, L)] = s1
            sc0, y0 = _rsqrt(s0)
            sc1, y1 = _rsqrt(s1)
            gbuf[r, pl.ds(0, L)] = LR * m0 / (sc0 * y0 + EPS)
            gbuf[r, pl.ds(L, L)] = LR * m1 / (sc1 * y1 + EPS)
            return carry
        lax.fori_loop(0, SB, _srow, 0)
        pltpu.sync_copy(rowbuf, out_state.at[idx_v.at[j]])

        pltpu.sync_copy(emb_hbm.at[idx_v.at[j]], rowbuf)

        def _erow(r, carry):
            rowbuf[r, pl.ds(0, L)] = (rowbuf[r, pl.ds(0, L)]
                                      - gbuf[r, pl.ds(0, L)])
            rowbuf[r, pl.ds(L, L)] = (rowbuf[r, pl.ds(L, L)]
                                      - gbuf[r, pl.ds(L, L)])
            return carry
        lax.fori_loop(0, SB, _erow, 0)
        pltpu.sync_copy(rowbuf, out_emb.at[idx_v.at[j]])


@functools.lru_cache(maxsize=1)
def _make_sc_update():
  return pl.kernel(
    _sc_body,
    out_type=(jax.ShapeDtypeStruct((M, D), jnp.float32),
              jax.ShapeDtypeStruct((M, D), jnp.float32)),
    mesh=plsc.VectorSubcoreMesh(
        core_axis_name="c", subcore_axis_name="s",
        num_cores=1, num_subcores=NS),
    scratch_types=[
        pltpu.VMEM_SHARED((M,), jnp.int32),
        pltpu.VMEM_SHARED((B, D), jnp.float32),
        pltpu.VMEM_SHARED((B, L), jnp.float32),
        pltpu.VMEM((NB, SB), jnp.int32),
        pltpu.VMEM((NB, SB), jnp.int32),
        pltpu.VMEM((SB, D), jnp.float32),
        pltpu.VMEM((SB, L), jnp.float32),
        pltpu.VMEM((SB, D), jnp.float32),
        pltpu.SemaphoreType.DMA,
        pltpu.SemaphoreType.DMA,
    ],
    compiler_params=pltpu.CompilerParams(use_tc_tiling_on_sc=False),
  )


def kernel(idx, grad, emb, state):
    idx2 = idx.reshape(B // SB, SB)
    new_emb, new_state = _make_sc_update()(idx2, grad, emb, state)
    return new_emb, new_state


# speedup vs baseline: 5.1410x; 5.1410x over previous
import functools

import jax
import jax.numpy as jnp
from jax import lax
from jax.experimental import pallas as pl
from jax.experimental.pallas import tpu as pltpu
from jax.experimental.pallas import tpu_sc as plsc
from jax._src.pallas import mpmd as _pl_mpmd

LR = 0.01
EPS = 1e-10
M = 1000000
D = 32
B = 16384

NS = 16
C = B // NS
SB = 128
NB = C // SB
L = 16


def _rsqrt(s):
    s = jnp.maximum(s, 1e-37)
    i = lax.bitcast_convert_type(s, jnp.int32)
    y = lax.bitcast_convert_type(jnp.int32(0x5F3759DF) - (i >> 1),
                                 jnp.float32)
    for _ in range(3):
        y = y * (1.5 - 0.5 * s * y * y)
    return s, y


def _mean_lanes(gbuf, cbuf, r):
    cnt = plsc.load_gather(cbuf, [jnp.full((L,), r, jnp.int32)])
    m0 = gbuf[r, pl.ds(0, L)] / cnt
    m1 = gbuf[r, pl.ds(L, L)] / cnt
    return m0, m1


def _sc_body(idx_hbm, grad_hbm, emb_al, state_al, out_emb, out_state,
             slot_tab, gsum, cnt1, hstage,
             idx_v, rep_v, gbuf, rowbuf, cbuf, sem):
    tid = lax.axis_index("s")
    base = tid * C

    zeros16 = jnp.zeros((L,), jnp.float32)
    ones16 = jnp.ones((L,), jnp.float32)
    iota16 = lax.iota(jnp.int32, L)

    pltpu.sync_copy(idx_hbm.at[pl.ds(tid * NB, NB)], idx_v)
    for j in range(NB):
        def _jfill(k, carry, j=j):
            rep_v[j, pl.ds(k * L, L)] = iota16 + (base + j * SB + k * L)
            return carry
        lax.fori_loop(0, SB // L, _jfill, 0)
        pltpu.sync_copy(rep_v.at[j], slot_tab.at[idx_v.at[j]])

    def _zrow(r, carry):
        gbuf[r, pl.ds(0, L)] = zeros16
        gbuf[r, pl.ds(L, L)] = zeros16
        return carry
    lax.fori_loop(0, SB, _zrow, 0)

    def _zc(k, carry):
        cbuf[pl.ds(k * L, L)] = zeros16
        return carry
    lax.fori_loop(0, SB // L, _zc, 0)

    for j in range(NB):
        pltpu.sync_copy(gbuf, gsum.at[pl.ds(base + j * SB, SB)])
        pltpu.sync_copy(cbuf, cnt1.at[pl.ds(base + j * SB, SB)])

    plsc.subcore_barrier()

    descs = [pltpu.async_copy(slot_tab.at[idx_v.at[j]], rep_v.at[j], sem)
             for j in range(NB)]
    for d in descs:
        d.wait()

    def _oc(k, carry):
        cbuf[pl.ds(k * L, L)] = ones16
        return carry
    lax.fori_loop(0, SB // L, _oc, 0)

    for j in range(NB):
        pltpu.sync_copy(grad_hbm.at[pl.ds(base + j * SB, SB)], gbuf)
        pltpu.sync_copy(gbuf, gsum.at[rep_v.at[j]], add=True)
        pltpu.sync_copy(cbuf, cnt1.at[rep_v.at[j]], add=True)

    plsc.subcore_barrier()

    for j in range(NB):
        pltpu.sync_copy(gsum.at[rep_v.at[j]], gbuf)
        pltpu.sync_copy(cnt1.at[rep_v.at[j]], cbuf)
        pltpu.sync_copy(state_al.at[idx_v.at[j]], rowbuf)

        def _srow(r, carry):
            m0, m1 = _mean_lanes(gbuf, cbuf, r)
            rowbuf[r, pl.ds(0, L)] = rowbuf[r, pl.ds(0, L)] + m0 * m0
            rowbuf[r, pl.ds(L, L)] = rowbuf[r, pl.ds(L, L)] + m1 * m1
            return carry
        lax.fori_loop(0, SB, _srow, 0)
        pltpu.sync_copy(rowbuf, hstage.at[pl.ds(base + j * SB, SB)])

    plsc.subcore_barrier()

    for j in range(NB):
        pltpu.sync_copy(hstage.at[pl.ds(base + j * SB, SB)], rowbuf)
        pltpu.sync_copy(rowbuf, out_state.at[idx_v.at[j]])

    plsc.subcore_barrier()

    for j in range(NB):
        pltpu.sync_copy(gsum.at[rep_v.at[j]], gbuf)
        pltpu.sync_copy(cnt1.at[rep_v.at[j]], cbuf)
        pltpu.sync_copy(out_state.at[idx_v.at[j]], rowbuf)

        def _trow(r, carry):
            m0, m1 = _mean_lanes(gbuf, cbuf, r)
            sc0, y0 = _rsqrt(rowbuf[r, pl.ds(0, L)])
            sc1, y1 = _rsqrt(rowbuf[r, pl.ds(L, L)])
            gbuf[r, pl.ds(0, L)] = LR * m0 / (sc0 * y0 + EPS)
            gbuf[r, pl.ds(L, L)] = LR * m1 / (sc1 * y1 + EPS)
            return carry
        lax.fori_loop(0, SB, _trow, 0)

        pltpu.sync_copy(emb_al.at[idx_v.at[j]], rowbuf)

        def _erow(r, carry):
            rowbuf[r, pl.ds(0, L)] = (rowbuf[r, pl.ds(0, L)]
                                      - gbuf[r, pl.ds(0, L)])
            rowbuf[r, pl.ds(L, L)] = (rowbuf[r, pl.ds(L, L)]
                                      - gbuf[r, pl.ds(L, L)])
            return carry
        lax.fori_loop(0, SB, _erow, 0)
        pltpu.sync_copy(rowbuf, hstage.at[pl.ds(base + j * SB, SB)])

    plsc.subcore_barrier()

    for j in range(NB):
        pltpu.sync_copy(hstage.at[pl.ds(base + j * SB, SB)], rowbuf)
        pltpu.sync_copy(rowbuf, out_emb.at[idx_v.at[j]])


@functools.lru_cache(maxsize=1)
def _make_sc_update():
  mesh = plsc.VectorSubcoreMesh(
      core_axis_name="c", subcore_axis_name="s",
      num_cores=1, num_subcores=NS)
  return _pl_mpmd._mpmd_map(
    [(mesh, _sc_body)],
    (jax.ShapeDtypeStruct((M, D), jnp.float32),
     jax.ShapeDtypeStruct((M, D), jnp.float32)),
    input_output_aliases={2: 0, 3: 1},
    scratch_types=[
        pltpu.VMEM_SHARED((M,), jnp.int32),
        pltpu.VMEM_SHARED((B, D), jnp.float32),
        pltpu.VMEM_SHARED((B,), jnp.float32),
        pltpu.MemorySpace.HBM((B, D), jnp.float32),
        pltpu.VMEM((NB, SB), jnp.int32),
        pltpu.VMEM((NB, SB), jnp.int32),
        pltpu.VMEM((SB, D), jnp.float32),
        pltpu.VMEM((SB, D), jnp.float32),
        pltpu.VMEM((SB,), jnp.float32),
        pltpu.SemaphoreType.DMA,
    ],
    compiler_params=pltpu.CompilerParams(use_tc_tiling_on_sc=False,
                                         needs_layout_passes=False),
  )


def kernel(idx, grad, emb, state):
    idx2 = idx.reshape(B // SB, SB)
    new_emb, new_state = _make_sc_update()(idx2, grad, emb, state)
    return new_emb, new_state


# speedup vs baseline: 5.1733x; 1.0063x over previous
import functools

import jax
import jax.numpy as jnp
from jax import lax
from jax.experimental import pallas as pl
from jax.experimental.pallas import tpu as pltpu
from jax.experimental.pallas import tpu_sc as plsc
from jax._src.pallas import mpmd as _pl_mpmd

LR = 0.01
EPS = 1e-10
M = 1000000
D = 32
B = 16384

NS = 16
C = B // NS
SB = 128
NB = C // SB
L = 16


def _rsqrt(s):
    s = jnp.maximum(s, 1e-37)
    i = lax.bitcast_convert_type(s, jnp.int32)
    y = lax.bitcast_convert_type(jnp.int32(0x5F3759DF) - (i >> 1),
                                 jnp.float32)
    for _ in range(3):
        y = y * (1.5 - 0.5 * s * y * y)
    return s, y


def _mean_lanes(gbuf, obuf, r):
    cnt = obuf[r]
    m0 = gbuf[r, pl.ds(0, L)] / cnt
    m1 = gbuf[r, pl.ds(L, L)] / cnt
    return m0, m1


def _sc_body(idx_hbm, grad_hbm, emb_al, state_al, out_emb, out_state,
             slot_tab, gsum, cntt, hstage,
             idx_v, rep_v, gbuf, rowbuf, obuf, sem):
    tid = lax.axis_index("s")
    base = tid * C

    zeros16 = jnp.zeros((L,), jnp.float32)
    ones16 = jnp.ones((L,), jnp.float32)
    iota16 = lax.iota(jnp.int32, L)

    pltpu.sync_copy(idx_hbm.at[pl.ds(tid * NB, NB)], idx_v)
    for j in range(NB):
        def _jfill(k, carry, j=j):
            rep_v[j, pl.ds(k * L, L)] = iota16 + (base + j * SB + k * L)
            return carry
        lax.fori_loop(0, SB // L, _jfill, 0)
        pltpu.sync_copy(rep_v.at[j], slot_tab.at[idx_v.at[j]])

    def _zrow(r, carry):
        gbuf[r, pl.ds(0, L)] = zeros16
        gbuf[r, pl.ds(L, L)] = zeros16
        return carry
    lax.fori_loop(0, SB, _zrow, 0)

    def _zc(r, carry):
        obuf[r] = zeros16
        return carry
    lax.fori_loop(0, SB, _zc, 0)

    for j in range(NB):
        pltpu.sync_copy(gbuf, gsum.at[pl.ds(base + j * SB, SB)])
        pltpu.sync_copy(obuf, cntt.at[pl.ds(base + j * SB, SB)])

    plsc.subcore_barrier()

    descs = [pltpu.async_copy(slot_tab.at[idx_v.at[j]], rep_v.at[j], sem)
             for j in range(NB)]
    for d in descs:
        d.wait()

    def _oc(r, carry):
        obuf[r] = ones16
        return carry
    lax.fori_loop(0, SB, _oc, 0)

    for j in range(NB):
        pltpu.sync_copy(grad_hbm.at[pl.ds(base + j * SB, SB)], gbuf)
        pltpu.sync_copy(gbuf, gsum.at[rep_v.at[j]], add=True)
        pltpu.sync_copy(obuf, cntt.at[rep_v.at[j]], add=True)

    plsc.subcore_barrier()

    for j in range(NB):
        pltpu.sync_copy(gsum.at[rep_v.at[j]], gbuf)
        pltpu.sync_copy(cntt.at[rep_v.at[j]], obuf)
        pltpu.sync_copy(state_al.at[idx_v.at[j]], rowbuf)

        def _srow(r, carry):
            m0, m1 = _mean_lanes(gbuf, obuf, r)
            rowbuf[r, pl.ds(0, L)] = rowbuf[r, pl.ds(0, L)] + m0 * m0
            rowbuf[r, pl.ds(L, L)] = rowbuf[r, pl.ds(L, L)] + m1 * m1
            return carry
        lax.fori_loop(0, SB, _srow, 0)
        pltpu.sync_copy(rowbuf, hstage.at[pl.ds(base + j * SB, SB)])

    plsc.subcore_barrier()

    for j in range(NB):
        pltpu.sync_copy(hstage.at[pl.ds(base + j * SB, SB)], rowbuf)
        pltpu.sync_copy(rowbuf, out_state.at[idx_v.at[j]])

    plsc.subcore_barrier()

    for j in range(NB):
        pltpu.sync_copy(gsum.at[rep_v.at[j]], gbuf)
        pltpu.sync_copy(cntt.at[rep_v.at[j]], obuf)
        pltpu.sync_copy(out_state.at[idx_v.at[j]], rowbuf)

        def _trow(r, carry):
            m0, m1 = _mean_lanes(gbuf, obuf, r)
            sc0, y0 = _rsqrt(rowbuf[r, pl.ds(0, L)])
            sc1, y1 = _rsqrt(rowbuf[r, pl.ds(L, L)])
            gbuf[r, pl.ds(0, L)] = LR * m0 / (sc0 * y0 + EPS)
            gbuf[r, pl.ds(L, L)] = LR * m1 / (sc1 * y1 + EPS)
            return carry
        lax.fori_loop(0, SB, _trow, 0)

        pltpu.sync_copy(emb_al.at[idx_v.at[j]], rowbuf)

        def _erow(r, carry):
            rowbuf[r, pl.ds(0, L)] = (rowbuf[r, pl.ds(0, L)]
                                      - gbuf[r, pl.ds(0, L)])
            rowbuf[r, pl.ds(L, L)] = (rowbuf[r, pl.ds(L, L)]
                                      - gbuf[r, pl.ds(L, L)])
            return carry
        lax.fori_loop(0, SB, _erow, 0)
        pltpu.sync_copy(rowbuf, hstage.at[pl.ds(base + j * SB, SB)])

    plsc.subcore_barrier()

    for j in range(NB):
        pltpu.sync_copy(hstage.at[pl.ds(base + j * SB, SB)], rowbuf)
        pltpu.sync_copy(rowbuf, out_emb.at[idx_v.at[j]])


@functools.lru_cache(maxsize=1)
def _make_sc_update():
  mesh = plsc.VectorSubcoreMesh(
      core_axis_name="c", subcore_axis_name="s",
      num_cores=1, num_subcores=NS)
  return _pl_mpmd._mpmd_map(
    [(mesh, _sc_body)],
    (jax.ShapeDtypeStruct((M, D), jnp.float32),
     jax.ShapeDtypeStruct((M, D), jnp.float32)),
    input_output_aliases={2: 0, 3: 1},
    scratch_types=[
        pltpu.VMEM_SHARED((M,), jnp.int32),
        pltpu.VMEM_SHARED((B, D), jnp.float32),
        pltpu.VMEM_SHARED((B, L), jnp.float32),
        pltpu.MemorySpace.HBM((B, D), jnp.float32),
        pltpu.VMEM((NB, SB), jnp.int32),
        pltpu.VMEM((NB, SB), jnp.int32),
        pltpu.VMEM((SB, D), jnp.float32),
        pltpu.VMEM((SB, D), jnp.float32),
        pltpu.VMEM((SB, L), jnp.float32),
        pltpu.SemaphoreType.DMA,
    ],
    compiler_params=pltpu.CompilerParams(use_tc_tiling_on_sc=False),
  )


def kernel(idx, grad, emb, state):
    idx2 = idx.reshape(B // SB, SB)
    new_emb, new_state = _make_sc_update()(idx2, grad, emb, state)
    return new_emb, new_state
